# Initial kernel scaffold; baseline (speedup 1.0000x reference)
import jax
import jax.numpy as jnp
from jax.experimental import pallas as pl


def kernel(user_emb, item_emb, edge_index):
    raise NotImplementedError("write your pallas kernel here")


# speedup vs baseline: 6.2303x; 6.2303x over previous
import functools

import jax
import jax.numpy as jnp
from jax import lax
from jax.experimental import pallas as pl
from jax.experimental.pallas import tpu as pltpu
from jax.experimental.pallas import tpu_sc as plsc

NU = 25000
NI = 25000
NN = NU + NI
D = 64
NE = 800000

NC = 2
NS = 16
LANES = 16

RPT = 1568
HPAD = NS * RPT
NPAD = NC * HPAD
PAD0 = HPAD - NU
TRASH = 512
ACC_ROWS = HPAD + TRASH
ZROWS = ACC_ROWS // NS

EC = 128
EPT = NE // NS
NCHUNK = -(-EPT // EC)
EPT_PAD = NCHUNK * EC
NE_PAD = NS * EPT_PAD

DR = 32
NDR = RPT // DR
ZC = 64

_f32 = jnp.float32
_i32 = jnp.int32

_mesh = plsc.VectorSubcoreMesh(core_axis_name="c", subcore_axis_name="s")
_params = pltpu.CompilerParams(needs_layout_passes=False,
                               use_tc_tiling_on_sc=False)


def _rsqrt(x):
    i = lax.bitcast_convert_type(x, _i32)
    i = jnp.int32(0x5F3759DF) - lax.shift_right_logical(i, 1)
    y = lax.bitcast_convert_type(i, _f32)
    for _ in range(3):
        y = y * (1.5 - 0.5 * x * y * y)
    return y


def _splat(buf, idx):
    return plsc.load_gather(buf, [jnp.broadcast_to(idx, (LANES,)).astype(_i32)])


@functools.partial(
    pl.kernel,
    out_type=(
        jax.ShapeDtypeStruct((NPAD,), _f32),
        jax.ShapeDtypeStruct((NPAD,), _f32),
        jax.ShapeDtypeStruct((NPAD,), _f32),
        jax.ShapeDtypeStruct((NPAD, D), _f32),
    ),
    mesh=_mesh,
    compiler_params=_params,
    scratch_types=(
        pltpu.MemorySpace.VMEM_SHARED((NPAD,), _f32),
        pltpu.MemorySpace.VMEM_SHARED((NPAD,), _f32),
        pltpu.VMEM((EC,), _i32),
        pltpu.VMEM((EC,), _i32),
        pltpu.VMEM((EC,), _i32),
        pltpu.VMEM((EC,), _i32),
        pltpu.VMEM((EC,), _f32),
        pltpu.VMEM((NPAD // NS,), _f32),
        pltpu.VMEM((RPT,), _f32),
        pltpu.VMEM((RPT,), _f32),
        pltpu.VMEM((RPT,), _f32),
        pltpu.VMEM((RPT,), _f32),
        pltpu.VMEM((RPT,), _f32),
        pltpu.VMEM((DR, D), _f32),
        pltpu.VMEM((DR, D), _f32),
    ),
)
def _k0(u0, src_hbm, dst_hbm, a1, b1, ab1, g0,
        degs, degd, srcb, dstb, srcpb, dstpb, onesb, zbuf,
        dslb, ddlb, aslb, bslb, abslb, u0c, gc):
    c = lax.axis_index("c")
    s = lax.axis_index("s")

    def zb(k, _):
        zbuf[pl.ds(k * LANES, LANES)] = jnp.zeros((LANES,), _f32)
        return 0
    lax.fori_loop(0, (NPAD // NS) // LANES, zb, 0)
    for k in range(EC // LANES):
        onesb[pl.ds(k * LANES, LANES)] = jnp.ones((LANES,), _f32)

    zoff = s * (NPAD // NS)
    pltpu.sync_copy(zbuf, degs.at[pl.ds(zoff, NPAD // NS)])
    pltpu.sync_copy(zbuf, degd.at[pl.ds(zoff, NPAD // NS)])
    plsc.subcore_barrier()

    def deg_step(i, _):
        base = s * EPT_PAD + i * EC
        pltpu.sync_copy(src_hbm.at[pl.ds(base, EC)], srcb)
        pltpu.sync_copy(dst_hbm.at[pl.ds(base, EC)], dstb)
        for j in range(EC // LANES):
            sl = pl.ds(j * LANES, LANES)
            sv = srcb[sl]
            srcpb[sl] = sv + jnp.where(sv >= NU, PAD0, 0)
            dv = dstb[sl]
            dstpb[sl] = dv + jnp.where(dv >= NU, PAD0, 0)
        pltpu.sync_copy(onesb, degs.at[srcpb], add=True)
        pltpu.sync_copy(onesb, degd.at[dstpb], add=True)
        return 0
    lax.fori_loop(0, NCHUNK, deg_step, 0)
    plsc.subcore_barrier()

    rb = c * HPAD + s * RPT
    pltpu.sync_copy(degs.at[pl.ds(rb, RPT)], dslb)
    pltpu.sync_copy(degd.at[pl.ds(rb, RPT)], ddlb)

    def scales(g, _):
        gsl = pl.ds(g * LANES, LANES)
        av = _rsqrt(dslb[gsl] + 1.0)
        bv = _rsqrt(ddlb[gsl] + 1.0)
        aslb[gsl] = av
        bslb[gsl] = bv
        abslb[gsl] = av * bv
        return 0
    lax.fori_loop(0, RPT // LANES, scales, 0)
    pltpu.sync_copy(aslb, a1.at[pl.ds(rb, RPT)])
    pltpu.sync_copy(bslb, b1.at[pl.ds(rb, RPT)])
    pltpu.sync_copy(abslb, ab1.at[pl.ds(rb, RPT)])

    def emit(k, _):
        r0 = k * DR
        pltpu.sync_copy(u0.at[pl.ds(rb + r0, DR)], u0c)
        for r in range(DR):
            sa = _splat(aslb, r0 + r)
            for q in range(D // LANES):
                qsl = pl.ds(q * LANES, LANES)
                gc[r, qsl] = u0c[r, qsl] * sa
        pltpu.sync_copy(gc, g0.at[pl.ds(rb + r0, DR)])
        return 0
    lax.fori_loop(0, NDR, emit, 0)


_LAYER_SCRATCH = (
    pltpu.MemorySpace.VMEM_SHARED((ACC_ROWS, D), _f32),
    pltpu.VMEM((EC,), _i32),
    pltpu.VMEM((EC,), _i32),
    pltpu.VMEM((EC,), _i32),
    pltpu.VMEM((EC,), _i32),
    pltpu.VMEM((EC, D), _f32),
    pltpu.VMEM((ZC, D), _f32),
)


def _zero_acc(acc, z2, s):
    def zb(k, _):
        r = k // (D // LANES)
        q = k % (D // LANES)
        z2[r, pl.ds(q * LANES, LANES)] = jnp.zeros((LANES,), _f32)
        return 0
    lax.fori_loop(0, ZC * (D // LANES), zb, 0)
    zbase = s * ZROWS

    def za(k, _):
        pltpu.sync_copy(z2, acc.at[pl.ds(zbase + k * ZC, ZC)])
        return 0
    lax.fori_loop(0, ZROWS // ZC, za, 0)


def _edge_phase(x, src_hbm, dst_hbm, acc, srcb, dstb, srcpb, dstlb,
                rows, sem, c, s):
    lo = c * NU

    def estep(i, _):
        base = s * EPT_PAD + i * EC
        pltpu.sync_copy(src_hbm.at[pl.ds(base, EC)], srcb)
        pltpu.sync_copy(dst_hbm.at[pl.ds(base, EC)], dstb)
        iot = lax.iota(_i32, LANES)
        for j in range(EC // LANES):
            sl = pl.ds(j * LANES, LANES)
            sv = srcb[sl]
            srcpb[sl] = sv + jnp.where(sv >= NU, PAD0, 0)
            dv = dstb[sl]
            inr = (dv >= lo) & (dv < lo + NU)
            tr = HPAD + (((i + j) * LANES + iot) & (TRASH - 1))
            dstlb[sl] = jnp.where(inr, dv - lo, tr)
        pltpu.async_copy(x.at[srcpb], rows, sem).wait()
        pltpu.sync_copy(rows, acc.at[dstlb], add=True)
        return 0
    lax.fori_loop(0, NCHUNK, estep, 0)


@functools.partial(
    pl.kernel,
    out_type=(
        jax.ShapeDtypeStruct((NPAD, D), _f32),
        jax.ShapeDtypeStruct((NPAD, D), _f32),
    ),
    mesh=_mesh,
    compiler_params=_params,
    scratch_types=_LAYER_SCRATCH + (
        pltpu.VMEM((RPT,), _f32),
        pltpu.VMEM((RPT,), _f32),
        pltpu.VMEM((DR, D), _f32),
        pltpu.VMEM((DR, D), _f32),
        pltpu.VMEM((DR, D), _f32),
        pltpu.SemaphoreType.DMA,
    ),
)
def _klayer(x, b1, ab1, src_hbm, dst_hbm, u_out, g_out,
            acc, srcb, dstb, srcpb, dstlb, rows, z2,
            bslb, abslb, accc, uc, gc2, sem):
    c = lax.axis_index("c")
    s = lax.axis_index("s")
    _zero_acc(acc, z2, s)
    plsc.subcore_barrier()
    _edge_phase(x, src_hbm, dst_hbm, acc, srcb, dstb, srcpb, dstlb,
                rows, sem, c, s)
    plsc.subcore_barrier()

    rb = c * HPAD + s * RPT
    la = s * RPT
    pltpu.sync_copy(b1.at[pl.ds(rb, RPT)], bslb)
    pltpu.sync_copy(ab1.at[pl.ds(rb, RPT)], abslb)

    def drain(k, _):
        r0 = k * DR
        pltpu.sync_copy(acc.at[pl.ds(la + r0, DR)], accc)
        for r in range(DR):
            sb = _splat(bslb, r0 + r)
            sab = _splat(abslb, r0 + r)
            for q in range(D // LANES):
                qsl = pl.ds(q * LANES, LANES)
                av = accc[r, qsl]
                uc[r, qsl] = av * sb
                gc2[r, qsl] = av * sab
        pltpu.sync_copy(uc, u_out.at[pl.ds(rb + r0, DR)])
        pltpu.sync_copy(gc2, g_out.at[pl.ds(rb + r0, DR)])
        return 0
    lax.fori_loop(0, NDR, drain, 0)


@functools.partial(
    pl.kernel,
    out_type=jax.ShapeDtypeStruct((NPAD, D), _f32),
    mesh=_mesh,
    compiler_params=_params,
    scratch_types=_LAYER_SCRATCH + (
        pltpu.VMEM((RPT,), _f32),
        pltpu.VMEM((DR, D), _f32),
        pltpu.VMEM((DR, D), _f32),
        pltpu.VMEM((DR, D), _f32),
        pltpu.VMEM((DR, D), _f32),
        pltpu.VMEM((DR, D), _f32),
        pltpu.SemaphoreType.DMA,
    ),
)
def _kfinal(x, b1, u0, u1, u2, src_hbm, dst_hbm, out,
            acc, srcb, dstb, srcpb, dstlb, rows, z2,
            bslb, accc, u0c, u1c, u2c, outc, sem):
    c = lax.axis_index("c")
    s = lax.axis_index("s")
    _zero_acc(acc, z2, s)
    plsc.subcore_barrier()
    _edge_phase(x, src_hbm, dst_hbm, acc, srcb, dstb, srcpb, dstlb,
                rows, sem, c, s)
    plsc.subcore_barrier()

    rb = c * HPAD + s * RPT
    la = s * RPT
    pltpu.sync_copy(b1.at[pl.ds(rb, RPT)], bslb)

    def drain(k, _):
        r0 = k * DR
        pltpu.sync_copy(acc.at[pl.ds(la + r0, DR)], accc)
        pltpu.sync_copy(u0.at[pl.ds(rb + r0, DR)], u0c)
        pltpu.sync_copy(u1.at[pl.ds(rb + r0, DR)], u1c)
        pltpu.sync_copy(u2.at[pl.ds(rb + r0, DR)], u2c)
        for r in range(DR):
            sb = _splat(bslb, r0 + r)
            for q in range(D // LANES):
                qsl = pl.ds(q * LANES, LANES)
                u3 = accc[r, qsl] * sb
                outc[r, qsl] = 0.25 * (u0c[r, qsl] + u1c[r, qsl]
                                       + u2c[r, qsl] + u3)
        pltpu.sync_copy(outc, out.at[pl.ds(rb + r0, DR)])
        return 0
    lax.fori_loop(0, NDR, drain, 0)


def kernel(user_emb, item_emb, edge_index):
    src = edge_index[0].astype(_i32)
    dst = edge_index[1].astype(_i32)
    padn = NE_PAD - NE
    pad = jnp.full((padn,), NN, _i32)
    srcp = jnp.concatenate([src, pad])
    dstp = jnp.concatenate([dst, pad])
    zpad = jnp.zeros((PAD0, D), _f32)
    u0 = jnp.concatenate([user_emb, zpad, item_emb, zpad], axis=0)

    a1, b1, ab1, g0 = _k0(u0, srcp, dstp)
    del a1
    u1, g1 = _klayer(g0, b1, ab1, srcp, dstp)
    u2, g2 = _klayer(g1, b1, ab1, srcp, dstp)
    out = _kfinal(g2, b1, u0, u1, u2, srcp, dstp)

    users = out[:NU]
    items = out[HPAD:HPAD + NI]
    return (users, items)


# speedup vs baseline: 9.4194x; 1.5119x over previous
import functools

import jax
import jax.numpy as jnp
from jax import lax
from jax.experimental import pallas as pl
from jax.experimental.pallas import tpu as pltpu
from jax.experimental.pallas import tpu_sc as plsc

NU = 25000
NI = 25000
NN = NU + NI
D = 64
NE = 800000

NC = 2
NS = 16
LANES = 16

RPT = 1568
HPAD = NS * RPT
NPAD = NC * HPAD
PAD0 = HPAD - NU
TRASH = 128
ACC_ROWS = HPAD + TRASH

EC = 128
NJ = 2
SUP = NJ * EC
EPT = NE // NS
NSUP = -(-EPT // SUP)
EPT_PAD = NSUP * SUP
NE_PAD = NS * EPT_PAD

DR = 32
NDR = RPT // DR
DRF = 16
NDRF = RPT // DRF
ZC = 32

_f32 = jnp.float32
_i32 = jnp.int32

_mesh = plsc.VectorSubcoreMesh(core_axis_name="c", subcore_axis_name="s")
_params = pltpu.CompilerParams(needs_layout_passes=False,
                               use_tc_tiling_on_sc=False)


def _rsqrt(x):
    i = lax.bitcast_convert_type(x, _i32)
    i = jnp.int32(0x5F3759DF) - lax.shift_right_logical(i, 1)
    y = lax.bitcast_convert_type(i, _f32)
    for _ in range(3):
        y = y * (1.5 - 0.5 * x * y * y)
    return y


def _splat(buf, idx):
    return plsc.load_gather(buf, [jnp.broadcast_to(idx, (LANES,)).astype(_i32)])


@functools.partial(
    pl.kernel,
    out_type=(
        jax.ShapeDtypeStruct((NPAD,), _f32),
        jax.ShapeDtypeStruct((NPAD,), _f32),
        jax.ShapeDtypeStruct((NPAD,), _f32),
        jax.ShapeDtypeStruct((NPAD, D), _f32),
    ),
    mesh=_mesh,
    compiler_params=_params,
    scratch_types=(
        pltpu.MemorySpace.VMEM_SHARED((NPAD,), _f32),
        pltpu.MemorySpace.VMEM_SHARED((NPAD,), _f32),
        pltpu.VMEM((SUP,), _i32),
        pltpu.VMEM((SUP,), _i32),
    ) + tuple(pltpu.VMEM((EC,), _i32) for _ in range(NJ))
      + tuple(pltpu.VMEM((EC,), _i32) for _ in range(NJ))
      + (
        pltpu.VMEM((EC,), _f32),
        pltpu.VMEM((NPAD // NS,), _f32),
        pltpu.VMEM((RPT,), _f32),
        pltpu.VMEM((RPT,), _f32),
        pltpu.VMEM((RPT,), _f32),
        pltpu.VMEM((RPT,), _f32),
        pltpu.VMEM((RPT,), _f32),
        pltpu.VMEM((DR, D), _f32),
        pltpu.VMEM((DR, D), _f32),
        pltpu.SemaphoreType.DMA((NJ,)),
        pltpu.SemaphoreType.DMA((NJ,)),
    ),
)
def _k0(u0, src_hbm, dst_hbm, a1, b1, ab1, g0, *scr):
    degs, degd, srcb, dstb = scr[0:4]
    spJ = scr[4:4 + NJ]
    dpJ = scr[4 + NJ:4 + 2 * NJ]
    (onesb, zbuf, dslb, ddlb, aslb, bslb, abslb, u0c, gc,
     semA, semB) = scr[4 + 2 * NJ:]
    c = lax.axis_index("c")
    s = lax.axis_index("s")

    def zb(k, _):
        zbuf[pl.ds(k * LANES, LANES)] = jnp.zeros((LANES,), _f32)
        return 0
    lax.fori_loop(0, (NPAD // NS) // LANES, zb, 0)
    for k in range(EC // LANES):
        onesb[pl.ds(k * LANES, LANES)] = jnp.ones((LANES,), _f32)

    zoff = s * (NPAD // NS)
    pltpu.sync_copy(zbuf, degs.at[pl.ds(zoff, NPAD // NS)])
    pltpu.sync_copy(zbuf, degd.at[pl.ds(zoff, NPAD // NS)])
    plsc.subcore_barrier()

    def deg_step(t, _):
        base = s * EPT_PAD + t * SUP
        pltpu.sync_copy(src_hbm.at[pl.ds(base, SUP)], srcb)
        pltpu.sync_copy(dst_hbm.at[pl.ds(base, SUP)], dstb)
        descs = []
        for j in range(NJ):
            for k in range(EC // LANES):
                sl = pl.ds(j * EC + k * LANES, LANES)
                kl = pl.ds(k * LANES, LANES)
                sv = srcb[sl]
                spJ[j][kl] = sv + jnp.where(sv >= NU, PAD0, 0)
                dv = dstb[sl]
                dpJ[j][kl] = dv + jnp.where(dv >= NU, PAD0, 0)
            descs.append(pltpu.async_copy(onesb, degs.at[spJ[j]],
                                          semA.at[j], add=True))
            descs.append(pltpu.async_copy(onesb, degd.at[dpJ[j]],
                                          semB.at[j], add=True))
        for d in descs:
            d.wait()
        return 0
    lax.fori_loop(0, NSUP, deg_step, 0)
    plsc.subcore_barrier()

    rb = c * HPAD + s * RPT
    pltpu.sync_copy(degs.at[pl.ds(rb, RPT)], dslb)
    pltpu.sync_copy(degd.at[pl.ds(rb, RPT)], ddlb)

    def scales(g, _):
        gsl = pl.ds(g * LANES, LANES)
        av = _rsqrt(dslb[gsl] + 1.0)
        bv = _rsqrt(ddlb[gsl] + 1.0)
        aslb[gsl] = av
        bslb[gsl] = bv
        abslb[gsl] = av * bv
        return 0
    lax.fori_loop(0, RPT // LANES, scales, 0)
    pltpu.sync_copy(aslb, a1.at[pl.ds(rb, RPT)])
    pltpu.sync_copy(bslb, b1.at[pl.ds(rb, RPT)])
    pltpu.sync_copy(abslb, ab1.at[pl.ds(rb, RPT)])

    def emit(k, _):
        r0 = k * DR
        pltpu.sync_copy(u0.at[pl.ds(rb + r0, DR)], u0c)
        for r in range(DR):
            sa = _splat(aslb, r0 + r)
            for q in range(D // LANES):
                qsl = pl.ds(q * LANES, LANES)
                gc[r, qsl] = u0c[r, qsl] * sa
        pltpu.sync_copy(gc, g0.at[pl.ds(rb + r0, DR)])
        return 0
    lax.fori_loop(0, NDR, emit, 0)


_LAYER_SCRATCH = (
    (pltpu.MemorySpace.VMEM_SHARED((ACC_ROWS, D), _f32),
     pltpu.VMEM((SUP,), _i32),
     pltpu.VMEM((SUP,), _i32))
    + tuple(pltpu.VMEM((EC,), _i32) for _ in range(NJ))
    + tuple(pltpu.VMEM((EC,), _i32) for _ in range(NJ))
    + tuple(pltpu.VMEM((EC, D), _f32) for _ in range(NJ))
    + (pltpu.VMEM((ZC, D), _f32),
       pltpu.SemaphoreType.DMA((NJ,)),
       pltpu.SemaphoreType.DMA((NJ,)))
)


def _zero_acc(acc, z2, s):
    def zb(k, _):
        r = k // (D // LANES)
        q = k % (D // LANES)
        z2[r, pl.ds(q * LANES, LANES)] = jnp.zeros((LANES,), _f32)
        return 0
    lax.fori_loop(0, ZC * (D // LANES), zb, 0)
    zbase = s * RPT

    def za(k, _):
        pltpu.sync_copy(z2, acc.at[pl.ds(zbase + k * ZC, ZC)])
        return 0
    lax.fori_loop(0, RPT // ZC, za, 0)


def _edge_phase(x, src_hbm, dst_hbm, acc, srcb, dstb, spJ, dlJ, rowsJ,
                semG, semS, c, s):
    lo = c * NU
    iot = lax.iota(_i32, LANES)

    def estep(t, _):
        base = s * EPT_PAD + t * SUP
        pltpu.sync_copy(src_hbm.at[pl.ds(base, SUP)], srcb)
        pltpu.sync_copy(dst_hbm.at[pl.ds(base, SUP)], dstb)
        gdescs = []
        for j in range(NJ):
            @pl.when(t > 0)
            def _(j=j):
                pltpu.make_async_copy(rowsJ[j], acc.at[dlJ[j]],
                                      semS.at[j]).wait()
            for k in range(EC // LANES):
                sl = pl.ds(j * EC + k * LANES, LANES)
                kl = pl.ds(k * LANES, LANES)
                sv = srcb[sl]
                spJ[j][kl] = sv + jnp.where(sv >= NU, PAD0, 0)
                dv = dstb[sl]
                inr = (dv >= lo) & (dv < lo + NU)
                tr = HPAD + ((j * EC + k * LANES + iot) & (TRASH - 1))
                dlJ[j][kl] = jnp.where(inr, dv - lo, tr)
            gdescs.append(pltpu.async_copy(x.at[spJ[j]], rowsJ[j],
                                           semG.at[j]))
        for j in range(NJ):
            gdescs[j].wait()
            pltpu.async_copy(rowsJ[j], acc.at[dlJ[j]], semS.at[j], add=True)
        return 0
    lax.fori_loop(0, NSUP, estep, 0)
    for j in range(NJ):
        pltpu.make_async_copy(rowsJ[j], acc.at[dlJ[j]], semS.at[j]).wait()


@functools.partial(
    pl.kernel,
    out_type=(
        jax.ShapeDtypeStruct((NPAD, D), _f32),
        jax.ShapeDtypeStruct((NPAD, D), _f32),
    ),
    mesh=_mesh,
    compiler_params=_params,
    scratch_types=_LAYER_SCRATCH + (
        pltpu.VMEM((RPT,), _f32),
        pltpu.VMEM((RPT,), _f32),
        pltpu.VMEM((DR, D), _f32),
        pltpu.VMEM((DR, D), _f32),
        pltpu.VMEM((DR, D), _f32),
    ),
)
def _klayer(x, b1, ab1, src_hbm, dst_hbm, u_out, g_out, *scr):
    acc, srcb, dstb = scr[0:3]
    spJ = scr[3:3 + NJ]
    dlJ = scr[3 + NJ:3 + 2 * NJ]
    rowsJ = scr[3 + 2 * NJ:3 + 3 * NJ]
    z2, semG, semS, bslb, abslb, accc, uc, gc2 = scr[3 + 3 * NJ:]
    c = lax.axis_index("c")
    s = lax.axis_index("s")
    _zero_acc(acc, z2, s)
    plsc.subcore_barrier()
    _edge_phase(x, src_hbm, dst_hbm, acc, srcb, dstb, spJ, dlJ, rowsJ,
                semG, semS, c, s)
    plsc.subcore_barrier()

    rb = c * HPAD + s * RPT
    la = s * RPT
    pltpu.sync_copy(b1.at[pl.ds(rb, RPT)], bslb)
    pltpu.sync_copy(ab1.at[pl.ds(rb, RPT)], abslb)

    def drain(k, _):
        r0 = k * DR
        pltpu.sync_copy(acc.at[pl.ds(la + r0, DR)], accc)
        for r in range(DR):
            sb = _splat(bslb, r0 + r)
            sab = _splat(abslb, r0 + r)
            for q in range(D // LANES):
                qsl = pl.ds(q * LANES, LANES)
                av = accc[r, qsl]
                uc[r, qsl] = av * sb
                gc2[r, qsl] = av * sab
        pltpu.sync_copy(uc, u_out.at[pl.ds(rb + r0, DR)])
        pltpu.sync_copy(gc2, g_out.at[pl.ds(rb + r0, DR)])
        return 0
    lax.fori_loop(0, NDR, drain, 0)


@functools.partial(
    pl.kernel,
    out_type=jax.ShapeDtypeStruct((NPAD, D), _f32),
    mesh=_mesh,
    compiler_params=_params,
    scratch_types=_LAYER_SCRATCH + (
        pltpu.VMEM((RPT,), _f32),
        pltpu.VMEM((DRF, D), _f32),
        pltpu.VMEM((DRF, D), _f32),
        pltpu.VMEM((DRF, D), _f32),
        pltpu.VMEM((DRF, D), _f32),
        pltpu.VMEM((DRF, D), _f32),
    ),
)
def _kfinal(x, b1, u0, u1, u2, src_hbm, dst_hbm, out, *scr):
    acc, srcb, dstb = scr[0:3]
    spJ = scr[3:3 + NJ]
    dlJ = scr[3 + NJ:3 + 2 * NJ]
    rowsJ = scr[3 + 2 * NJ:3 + 3 * NJ]
    z2, semG, semS, bslb, accc, u0c, u1c, u2c, outc = scr[3 + 3 * NJ:]
    c = lax.axis_index("c")
    s = lax.axis_index("s")
    _zero_acc(acc, z2, s)
    plsc.subcore_barrier()
    _edge_phase(x, src_hbm, dst_hbm, acc, srcb, dstb, spJ, dlJ, rowsJ,
                semG, semS, c, s)
    plsc.subcore_barrier()

    rb = c * HPAD + s * RPT
    la = s * RPT
    pltpu.sync_copy(b1.at[pl.ds(rb, RPT)], bslb)

    def drain(k, _):
        r0 = k * DRF
        pltpu.sync_copy(acc.at[pl.ds(la + r0, DRF)], accc)
        pltpu.sync_copy(u0.at[pl.ds(rb + r0, DRF)], u0c)
        pltpu.sync_copy(u1.at[pl.ds(rb + r0, DRF)], u1c)
        pltpu.sync_copy(u2.at[pl.ds(rb + r0, DRF)], u2c)
        for r in range(DRF):
            sb = _splat(bslb, r0 + r)
            for q in range(D // LANES):
                qsl = pl.ds(q * LANES, LANES)
                u3 = accc[r, qsl] * sb
                outc[r, qsl] = 0.25 * (u0c[r, qsl] + u1c[r, qsl]
                                       + u2c[r, qsl] + u3)
        pltpu.sync_copy(outc, out.at[pl.ds(rb + r0, DRF)])
        return 0
    lax.fori_loop(0, NDRF, drain, 0)


def kernel(user_emb, item_emb, edge_index):
    src = edge_index[0].astype(_i32)
    dst = edge_index[1].astype(_i32)
    padn = NE_PAD - NE
    pad = jnp.full((padn,), NN, _i32)
    srcp = jnp.concatenate([src, pad])
    dstp = jnp.concatenate([dst, pad])
    zpad = jnp.zeros((PAD0, D), _f32)
    u0 = jnp.concatenate([user_emb, zpad, item_emb, zpad], axis=0)

    a1, b1, ab1, g0 = _k0(u0, srcp, dstp)
    del a1
    u1, g1 = _klayer(g0, b1, ab1, srcp, dstp)
    u2, g2 = _klayer(g1, b1, ab1, srcp, dstp)
    out = _kfinal(g2, b1, u0, u1, u2, srcp, dstp)

    users = out[:NU]
    items = out[HPAD:HPAD + NI]
    return (users, items)


# speedup vs baseline: 11.8571x; 1.2588x over previous
import functools

import jax
import jax.numpy as jnp
from jax import lax
from jax.experimental import pallas as pl
from jax.experimental.pallas import tpu as pltpu
from jax.experimental.pallas import tpu_sc as plsc

NU = 25000
NI = 25000
NN = NU + NI
D = 64
NE = 800000

NC = 2
NS = 16
LANES = 16

RPT = 1568
HPAD = NS * RPT
NPAD = NC * HPAD
PAD0 = HPAD - NU
TRASH = 128
ACC_ROWS = HPAD + TRASH

EC = 128
NJ = 2
SUP = NJ * EC
EPT = NE // NS
NSUP = -(-EPT // SUP)
EPT_PAD = NSUP * SUP
NE_PAD = NS * EPT_PAD
STG = SUP + LANES

DR = 32
NDR = RPT // DR
DRF = 16
NDRF = RPT // DRF
ZC = 32

_f32 = jnp.float32
_i32 = jnp.int32

_mesh = plsc.VectorSubcoreMesh(core_axis_name="c", subcore_axis_name="s")
_params = pltpu.CompilerParams(needs_layout_passes=False,
                               use_tc_tiling_on_sc=False)


def _rsqrt(x):
    i = lax.bitcast_convert_type(x, _i32)
    i = jnp.int32(0x5F3759DF) - lax.shift_right_logical(i, 1)
    y = lax.bitcast_convert_type(i, _f32)
    for _ in range(3):
        y = y * (1.5 - 0.5 * x * y * y)
    return y


def _splat(buf, idx):
    return plsc.load_gather(buf, [jnp.broadcast_to(idx, (LANES,)).astype(_i32)])


@functools.partial(
    pl.kernel,
    out_type=(
        jax.ShapeDtypeStruct((NPAD,), _f32),
        jax.ShapeDtypeStruct((NPAD,), _f32),
        jax.ShapeDtypeStruct((NPAD,), _f32),
        jax.ShapeDtypeStruct((NPAD, D), _f32),
        jax.ShapeDtypeStruct((NC * NS * EPT_PAD,), _i32),
        jax.ShapeDtypeStruct((NC * NS * EPT_PAD,), _i32),
        jax.ShapeDtypeStruct((NC * NS * LANES,), _i32),
    ),
    mesh=_mesh,
    compiler_params=_params,
    scratch_types=(
        pltpu.MemorySpace.VMEM_SHARED((NPAD,), _f32),
        pltpu.MemorySpace.VMEM_SHARED((NPAD,), _f32),
        pltpu.VMEM((SUP,), _i32),
        pltpu.VMEM((SUP,), _i32),
    ) + tuple(pltpu.VMEM((EC,), _i32) for _ in range(NJ))
      + tuple(pltpu.VMEM((EC,), _i32) for _ in range(NJ))
      + (
        pltpu.VMEM((STG,), _i32),
        pltpu.VMEM((STG,), _i32),
        pltpu.VMEM((LANES,), _i32),
        pltpu.VMEM((EC,), _f32),
        pltpu.VMEM((NPAD // NS,), _f32),
        pltpu.VMEM((RPT,), _f32),
        pltpu.VMEM((RPT,), _f32),
        pltpu.VMEM((RPT,), _f32),
        pltpu.VMEM((RPT,), _f32),
        pltpu.VMEM((RPT,), _f32),
        pltpu.VMEM((DR, D), _f32),
        pltpu.VMEM((DR, D), _f32),
        pltpu.SemaphoreType.DMA((NJ,)),
        pltpu.SemaphoreType.DMA((NJ,)),
    ),
)
def _k0(u0, src_hbm, dst_hbm, a1, b1, ab1, g0, srcPart, dstPart, blocks,
        *scr):
    degs, degd, srcb, dstb = scr[0:4]
    spJ = scr[4:4 + NJ]
    dpJ = scr[4 + NJ:4 + 2 * NJ]
    (stgS, stgD, blkb, onesb, zbuf, dslb, ddlb, aslb, bslb, abslb,
     u0c, gc, semA, semB) = scr[4 + 2 * NJ:]
    c = lax.axis_index("c")
    s = lax.axis_index("s")
    lo = c * NU
    iot = lax.iota(_i32, LANES)
    pbase = (c * NS + s) * EPT_PAD

    def zb(k, _):
        zbuf[pl.ds(k * LANES, LANES)] = jnp.zeros((LANES,), _f32)
        return 0
    lax.fori_loop(0, (NPAD // NS) // LANES, zb, 0)
    for k in range(EC // LANES):
        onesb[pl.ds(k * LANES, LANES)] = jnp.ones((LANES,), _f32)

    zoff = s * (NPAD // NS)
    pltpu.sync_copy(zbuf, degs.at[pl.ds(zoff, NPAD // NS)])
    pltpu.sync_copy(zbuf, degd.at[pl.ds(zoff, NPAD // NS)])
    plsc.subcore_barrier()

    def deg_step(t, carry):
        off, wp = carry
        base = s * EPT_PAD + t * SUP
        pltpu.sync_copy(src_hbm.at[pl.ds(base, SUP)], srcb)
        pltpu.sync_copy(dst_hbm.at[pl.ds(base, SUP)], dstb)
        descs = []
        for j in range(NJ):
            for k in range(EC // LANES):
                sl = pl.ds(j * EC + k * LANES, LANES)
                kl = pl.ds(k * LANES, LANES)
                sv = srcb[sl]
                spv = sv + jnp.where(sv >= NU, PAD0, 0)
                spJ[j][kl] = spv
                dv = dstb[sl]
                dpJ[j][kl] = dv + jnp.where(dv >= NU, PAD0, 0)
                m = (dv >= lo) & (dv < lo + NU)
                plsc.store_compressed(stgS.at[pl.ds(off, LANES)], spv, mask=m)
                plsc.store_compressed(stgD.at[pl.ds(off, LANES)], dv - lo, mask=m)
                off = off + jnp.sum(m.astype(_i32))

                @pl.when(off >= SUP)
                def _(wp=wp):
                    pltpu.sync_copy(stgS.at[pl.ds(0, SUP)],
                                    srcPart.at[pl.ds(pbase + wp * SUP, SUP)])
                    pltpu.sync_copy(stgD.at[pl.ds(0, SUP)],
                                    dstPart.at[pl.ds(pbase + wp * SUP, SUP)])
                    ts = stgS[pl.ds(SUP, LANES)]
                    stgS[pl.ds(0, LANES)] = ts
                    td = stgD[pl.ds(SUP, LANES)]
                    stgD[pl.ds(0, LANES)] = td
                wp = wp + jnp.where(off >= SUP, 1, 0)
                off = jnp.where(off >= SUP, off - SUP, off)
            descs.append(pltpu.async_copy(onesb, degs.at[spJ[j]],
                                          semA.at[j], add=True))
            descs.append(pltpu.async_copy(onesb, degd.at[dpJ[j]],
                                          semB.at[j], add=True))
        for d in descs:
            d.wait()
        return (off, wp)
    off, wp = lax.fori_loop(0, NSUP, deg_step,
                            (jnp.int32(0), jnp.int32(0)))

    @pl.when(off > 0)
    def _():
        for g in range(STG // LANES):
            sl = pl.ds(g * LANES, LANES)
            p = g * LANES + iot
            mm = p >= off
            vs = stgS[sl]
            stgS[sl] = jnp.where(mm, NN + PAD0, vs)
            vd = stgD[sl]
            stgD[sl] = jnp.where(mm, HPAD + (p & (TRASH - 1)), vd)
        pltpu.sync_copy(stgS.at[pl.ds(0, SUP)],
                        srcPart.at[pl.ds(pbase + wp * SUP, SUP)])
        pltpu.sync_copy(stgD.at[pl.ds(0, SUP)],
                        dstPart.at[pl.ds(pbase + wp * SUP, SUP)])
    nblk = wp + jnp.where(off > 0, 1, 0)
    blkb[pl.ds(0, LANES)] = jnp.broadcast_to(nblk, (LANES,)).astype(_i32)
    pltpu.sync_copy(blkb, blocks.at[pl.ds((c * NS + s) * LANES, LANES)])
    plsc.subcore_barrier()

    rb = c * HPAD + s * RPT
    pltpu.sync_copy(degs.at[pl.ds(rb, RPT)], dslb)
    pltpu.sync_copy(degd.at[pl.ds(rb, RPT)], ddlb)

    def scales(g, _):
        gsl = pl.ds(g * LANES, LANES)
        av = _rsqrt(dslb[gsl] + 1.0)
        bv = _rsqrt(ddlb[gsl] + 1.0)
        aslb[gsl] = av
        bslb[gsl] = bv
        abslb[gsl] = av * bv
        return 0
    lax.fori_loop(0, RPT // LANES, scales, 0)
    pltpu.sync_copy(aslb, a1.at[pl.ds(rb, RPT)])
    pltpu.sync_copy(bslb, b1.at[pl.ds(rb, RPT)])
    pltpu.sync_copy(abslb, ab1.at[pl.ds(rb, RPT)])

    def emit(k, _):
        r0 = k * DR
        pltpu.sync_copy(u0.at[pl.ds(rb + r0, DR)], u0c)
        for r in range(DR):
            sa = _splat(aslb, r0 + r)
            for q in range(D // LANES):
                qsl = pl.ds(q * LANES, LANES)
                gc[r, qsl] = u0c[r, qsl] * sa
        pltpu.sync_copy(gc, g0.at[pl.ds(rb + r0, DR)])
        return 0
    lax.fori_loop(0, NDR, emit, 0)


_LAYER_SCRATCH = (
    (pltpu.MemorySpace.VMEM_SHARED((ACC_ROWS, D), _f32),)
    + tuple(pltpu.VMEM((EC,), _i32) for _ in range(NJ))
    + tuple(pltpu.VMEM((EC,), _i32) for _ in range(NJ))
    + tuple(pltpu.VMEM((EC, D), _f32) for _ in range(NJ))
    + (pltpu.VMEM((ZC, D), _f32),
       pltpu.VMEM((LANES,), _i32),
       pltpu.SemaphoreType.DMA((NJ,)),
       pltpu.SemaphoreType.DMA((NJ,)))
)


def _zero_acc(acc, z2, s):
    def zb(k, _):
        r = k // (D // LANES)
        q = k % (D // LANES)
        z2[r, pl.ds(q * LANES, LANES)] = jnp.zeros((LANES,), _f32)
        return 0
    lax.fori_loop(0, ZC * (D // LANES), zb, 0)
    zbase = s * RPT

    def za(k, _):
        pltpu.sync_copy(z2, acc.at[pl.ds(zbase + k * ZC, ZC)])
        return 0
    lax.fori_loop(0, RPT // ZC, za, 0)


def _edge_phase(x, srcPart, dstPart, blocks, acc, spJ, dlJ, rowsJ, blkb,
                semG, semS, c, s):
    pbase = (c * NS + s) * EPT_PAD
    pltpu.sync_copy(blocks.at[pl.ds((c * NS + s) * LANES, LANES)], blkb)
    nb = jnp.max(blkb[pl.ds(0, LANES)])

    def estep(t, _):
        base = pbase + t * SUP
        gdescs = []
        for j in range(NJ):
            @pl.when(t > 0)
            def _(j=j):
                pltpu.make_async_copy(rowsJ[j], acc.at[dlJ[j]],
                                      semS.at[j]).wait()
            pltpu.sync_copy(srcPart.at[pl.ds(base + j * EC, EC)], spJ[j])
            pltpu.sync_copy(dstPart.at[pl.ds(base + j * EC, EC)], dlJ[j])
            gdescs.append(pltpu.async_copy(x.at[spJ[j]], rowsJ[j],
                                           semG.at[j]))
        for j in range(NJ):
            gdescs[j].wait()
            pltpu.async_copy(rowsJ[j], acc.at[dlJ[j]], semS.at[j], add=True)
        return 0
    lax.fori_loop(0, nb, estep, 0)

    @pl.when(nb > 0)
    def _():
        for j in range(NJ):
            pltpu.make_async_copy(rowsJ[j], acc.at[dlJ[j]],
                                  semS.at[j]).wait()


@functools.partial(
    pl.kernel,
    out_type=(
        jax.ShapeDtypeStruct((NPAD, D), _f32),
        jax.ShapeDtypeStruct((NPAD, D), _f32),
    ),
    mesh=_mesh,
    compiler_params=_params,
    scratch_types=_LAYER_SCRATCH + (
        pltpu.VMEM((RPT,), _f32),
        pltpu.VMEM((RPT,), _f32),
        pltpu.VMEM((DR, D), _f32),
        pltpu.VMEM((DR, D), _f32),
        pltpu.VMEM((DR, D), _f32),
    ),
)
def _klayer(x, b1, ab1, srcPart, dstPart, blocks, u_out, g_out, *scr):
    acc = scr[0]
    spJ = scr[1:1 + NJ]
    dlJ = scr[1 + NJ:1 + 2 * NJ]
    rowsJ = scr[1 + 2 * NJ:1 + 3 * NJ]
    z2, blkb, semG, semS, bslb, abslb, accc, uc, gc2 = scr[1 + 3 * NJ:]
    c = lax.axis_index("c")
    s = lax.axis_index("s")
    _zero_acc(acc, z2, s)
    plsc.subcore_barrier()
    _edge_phase(x, srcPart, dstPart, blocks, acc, spJ, dlJ, rowsJ, blkb,
                semG, semS, c, s)
    plsc.subcore_barrier()

    rb = c * HPAD + s * RPT
    la = s * RPT
    pltpu.sync_copy(b1.at[pl.ds(rb, RPT)], bslb)
    pltpu.sync_copy(ab1.at[pl.ds(rb, RPT)], abslb)

    def drain(k, _):
        r0 = k * DR
        pltpu.sync_copy(acc.at[pl.ds(la + r0, DR)], accc)
        for r in range(DR):
            sb = _splat(bslb, r0 + r)
            sab = _splat(abslb, r0 + r)
            for q in range(D // LANES):
                qsl = pl.ds(q * LANES, LANES)
                av = accc[r, qsl]
                uc[r, qsl] = av * sb
                gc2[r, qsl] = av * sab
        pltpu.sync_copy(uc, u_out.at[pl.ds(rb + r0, DR)])
        pltpu.sync_copy(gc2, g_out.at[pl.ds(rb + r0, DR)])
        return 0
    lax.fori_loop(0, NDR, drain, 0)


@functools.partial(
    pl.kernel,
    out_type=jax.ShapeDtypeStruct((NPAD, D), _f32),
    mesh=_mesh,
    compiler_params=_params,
    scratch_types=_LAYER_SCRATCH + (
        pltpu.VMEM((RPT,), _f32),
        pltpu.VMEM((DRF, D), _f32),
        pltpu.VMEM((DRF, D), _f32),
        pltpu.VMEM((DRF, D), _f32),
        pltpu.VMEM((DRF, D), _f32),
        pltpu.VMEM((DRF, D), _f32),
    ),
)
def _kfinal(x, b1, u0, u1, u2, srcPart, dstPart, blocks, out, *scr):
    acc = scr[0]
    spJ = scr[1:1 + NJ]
    dlJ = scr[1 + NJ:1 + 2 * NJ]
    rowsJ = scr[1 + 2 * NJ:1 + 3 * NJ]
    z2, blkb, semG, semS, bslb, accc, u0c, u1c, u2c, outc = scr[1 + 3 * NJ:]
    c = lax.axis_index("c")
    s = lax.axis_index("s")
    _zero_acc(acc, z2, s)
    plsc.subcore_barrier()
    _edge_phase(x, srcPart, dstPart, blocks, acc, spJ, dlJ, rowsJ, blkb,
                semG, semS, c, s)
    plsc.subcore_barrier()

    rb = c * HPAD + s * RPT
    la = s * RPT
    pltpu.sync_copy(b1.at[pl.ds(rb, RPT)], bslb)

    def drain(k, _):
        r0 = k * DRF
        pltpu.sync_copy(acc.at[pl.ds(la + r0, DRF)], accc)
        pltpu.sync_copy(u0.at[pl.ds(rb + r0, DRF)], u0c)
        pltpu.sync_copy(u1.at[pl.ds(rb + r0, DRF)], u1c)
        pltpu.sync_copy(u2.at[pl.ds(rb + r0, DRF)], u2c)
        for r in range(DRF):
            sb = _splat(bslb, r0 + r)
            for q in range(D // LANES):
                qsl = pl.ds(q * LANES, LANES)
                u3 = accc[r, qsl] * sb
                outc[r, qsl] = 0.25 * (u0c[r, qsl] + u1c[r, qsl]
                                       + u2c[r, qsl] + u3)
        pltpu.sync_copy(outc, out.at[pl.ds(rb + r0, DRF)])
        return 0
    lax.fori_loop(0, NDRF, drain, 0)


def kernel(user_emb, item_emb, edge_index):
    src = edge_index[0].astype(_i32)
    dst = edge_index[1].astype(_i32)
    padn = NE_PAD - NE
    pad = jnp.full((padn,), NN, _i32)
    srcp = jnp.concatenate([src, pad])
    dstp = jnp.concatenate([dst, pad])
    zpad = jnp.zeros((PAD0, D), _f32)
    u0 = jnp.concatenate([user_emb, zpad, item_emb, zpad], axis=0)

    a1, b1, ab1, g0, srcPart, dstPart, blocks = _k0(u0, srcp, dstp)
    del a1
    u1, g1 = _klayer(g0, b1, ab1, srcPart, dstPart, blocks)
    u2, g2 = _klayer(g1, b1, ab1, srcPart, dstPart, blocks)
    out = _kfinal(g2, b1, u0, u1, u2, srcPart, dstPart, blocks)

    users = out[:NU]
    items = out[HPAD:HPAD + NI]
    return (users, items)


# speedup vs baseline: 12.1025x; 1.0207x over previous
import functools

import jax
import jax.numpy as jnp
from jax import lax
from jax.experimental import pallas as pl
from jax.experimental.pallas import tpu as pltpu
from jax.experimental.pallas import tpu_sc as plsc

NU = 25000
NI = 25000
NN = NU + NI
D = 64
NE = 800000

NC = 2
NS = 16
LANES = 16

RPT = 1568
HPAD = NS * RPT
NPAD = NC * HPAD
PAD0 = HPAD - NU
TRASH = 128
ACC_ROWS = HPAD + TRASH

EC = 128
NJ = 2
SUP = NJ * EC
EPT = NE // NS
NSUP = -(-EPT // SUP)
EPT_PAD = NSUP * SUP
NE_PAD = NS * EPT_PAD
STG = SUP + LANES

DR = 32
NDR = RPT // DR
DRF = 16
NDRF = RPT // DRF
ZC = 32

_f32 = jnp.float32
_i32 = jnp.int32

_mesh = plsc.VectorSubcoreMesh(core_axis_name="c", subcore_axis_name="s")
_params = pltpu.CompilerParams(needs_layout_passes=False,
                               use_tc_tiling_on_sc=False)


def _rsqrt(x):
    i = lax.bitcast_convert_type(x, _i32)
    i = jnp.int32(0x5F3759DF) - lax.shift_right_logical(i, 1)
    y = lax.bitcast_convert_type(i, _f32)
    for _ in range(3):
        y = y * (1.5 - 0.5 * x * y * y)
    return y


def _splat(buf, idx):
    return plsc.load_gather(buf, [jnp.broadcast_to(idx, (LANES,)).astype(_i32)])


@functools.partial(
    pl.kernel,
    out_type=(
        jax.ShapeDtypeStruct((NPAD,), _f32),
        jax.ShapeDtypeStruct((NPAD,), _f32),
        jax.ShapeDtypeStruct((NPAD,), _f32),
        jax.ShapeDtypeStruct((NPAD, D), _f32),
        jax.ShapeDtypeStruct((NC * NS * EPT_PAD,), _i32),
        jax.ShapeDtypeStruct((NC * NS * EPT_PAD,), _i32),
        jax.ShapeDtypeStruct((NC * NS * LANES,), _i32),
    ),
    mesh=_mesh,
    compiler_params=_params,
    scratch_types=(
        pltpu.MemorySpace.VMEM_SHARED((NPAD,), _f32),
        pltpu.MemorySpace.VMEM_SHARED((NPAD,), _f32),
        pltpu.VMEM((SUP,), _i32),
        pltpu.VMEM((SUP,), _i32),
    ) + tuple(pltpu.VMEM((EC,), _i32) for _ in range(NJ))
      + tuple(pltpu.VMEM((EC,), _i32) for _ in range(NJ))
      + (
        pltpu.VMEM((2 * STG,), _i32),
        pltpu.VMEM((2 * STG,), _i32),
        pltpu.VMEM((LANES,), _i32),
        pltpu.VMEM((EC,), _f32),
        pltpu.VMEM((NPAD // NS,), _f32),
        pltpu.VMEM((RPT,), _f32),
        pltpu.VMEM((RPT,), _f32),
        pltpu.VMEM((RPT,), _f32),
        pltpu.VMEM((RPT,), _f32),
        pltpu.VMEM((RPT,), _f32),
        pltpu.VMEM((DR, D), _f32),
        pltpu.VMEM((DR, D), _f32),
        pltpu.SemaphoreType.DMA((NJ,)),
        pltpu.SemaphoreType.DMA((NJ,)),
        pltpu.SemaphoreType.DMA((2,)),
    ),
)
def _k0(u0, src_hbm, dst_hbm, a1, b1, ab1, g0, srcPart, dstPart, blocks,
        *scr):
    degs, degd, srcb, dstb = scr[0:4]
    spJ = scr[4:4 + NJ]
    dpJ = scr[4 + NJ:4 + 2 * NJ]
    (stgS, stgD, blkb, onesb, zbuf, dslb, ddlb, aslb, bslb, abslb,
     u0c, gc, semA, semB, semF) = scr[4 + 2 * NJ:]
    c = lax.axis_index("c")
    s = lax.axis_index("s")
    lo = c * NU
    iot = lax.iota(_i32, LANES)
    pbase = (c * NS + s) * EPT_PAD

    def zb(k, _):
        zbuf[pl.ds(k * LANES, LANES)] = jnp.zeros((LANES,), _f32)
        return 0
    lax.fori_loop(0, (NPAD // NS) // LANES, zb, 0)
    for k in range(EC // LANES):
        onesb[pl.ds(k * LANES, LANES)] = jnp.ones((LANES,), _f32)

    zoff = s * (NPAD // NS)
    pltpu.sync_copy(zbuf, degs.at[pl.ds(zoff, NPAD // NS)])
    pltpu.sync_copy(zbuf, degd.at[pl.ds(zoff, NPAD // NS)])
    plsc.subcore_barrier()

    def deg_step(t, carry):
        off, wp = carry
        base = s * EPT_PAD + t * SUP
        pltpu.sync_copy(src_hbm.at[pl.ds(base, SUP)], srcb)
        pltpu.sync_copy(dst_hbm.at[pl.ds(base, SUP)], dstb)
        for j in range(NJ):
            @pl.when(t > 0)
            def _(j=j):
                pltpu.make_async_copy(onesb, degs.at[spJ[j]],
                                      semA.at[j]).wait()
                pltpu.make_async_copy(onesb, degd.at[dpJ[j]],
                                      semB.at[j]).wait()
            for k in range(EC // LANES):
                sl = pl.ds(j * EC + k * LANES, LANES)
                kl = pl.ds(k * LANES, LANES)
                sv = srcb[sl]
                spv = sv + jnp.where(sv >= NU, PAD0, 0)
                spJ[j][kl] = spv
                dv = dstb[sl]
                dpJ[j][kl] = dv + jnp.where(dv >= NU, PAD0, 0)
                sb2 = (wp & 1) * STG
                m = (dv >= lo) & (dv < lo + NU)
                plsc.store_compressed(stgS.at[pl.ds(sb2 + off, LANES)], spv,
                                      mask=m)
                plsc.store_compressed(stgD.at[pl.ds(sb2 + off, LANES)],
                                      dv - lo, mask=m)
                off = off + jnp.sum(m.astype(_i32))
                fl = off >= SUP

                @pl.when(fl & (wp >= 1))
                def _():
                    pltpu.make_async_copy(
                        stgS.at[pl.ds(0, SUP)],
                        srcPart.at[pl.ds(pbase, SUP)], semF.at[0]).wait()
                    pltpu.make_async_copy(
                        stgD.at[pl.ds(0, SUP)],
                        dstPart.at[pl.ds(pbase, SUP)], semF.at[1]).wait()

                @pl.when(fl)
                def _(wp=wp, sb2=sb2):
                    pltpu.async_copy(stgS.at[pl.ds(sb2, SUP)],
                                     srcPart.at[pl.ds(pbase + wp * SUP, SUP)],
                                     semF.at[0])
                    pltpu.async_copy(stgD.at[pl.ds(sb2, SUP)],
                                     dstPart.at[pl.ds(pbase + wp * SUP, SUP)],
                                     semF.at[1])
                    ob = ((wp + 1) & 1) * STG
                    ts = stgS[pl.ds(sb2 + SUP, LANES)]
                    stgS[pl.ds(ob, LANES)] = ts
                    td = stgD[pl.ds(sb2 + SUP, LANES)]
                    stgD[pl.ds(ob, LANES)] = td
                wp = wp + jnp.where(fl, 1, 0)
                off = jnp.where(fl, off - SUP, off)
            pltpu.async_copy(onesb, degs.at[spJ[j]], semA.at[j], add=True)
            pltpu.async_copy(onesb, degd.at[dpJ[j]], semB.at[j], add=True)
        return (off, wp)
    off, wp = lax.fori_loop(0, NSUP, deg_step,
                            (jnp.int32(0), jnp.int32(0)))

    for j in range(NJ):
        pltpu.make_async_copy(onesb, degs.at[spJ[j]], semA.at[j]).wait()
        pltpu.make_async_copy(onesb, degd.at[dpJ[j]], semB.at[j]).wait()

    @pl.when(wp >= 1)
    def _():
        pltpu.make_async_copy(stgS.at[pl.ds(0, SUP)],
                              srcPart.at[pl.ds(pbase, SUP)],
                              semF.at[0]).wait()
        pltpu.make_async_copy(stgD.at[pl.ds(0, SUP)],
                              dstPart.at[pl.ds(pbase, SUP)],
                              semF.at[1]).wait()

    @pl.when(off > 0)
    def _():
        fb = (wp & 1) * STG
        for g in range(STG // LANES):
            sl = pl.ds(fb + g * LANES, LANES)
            p = g * LANES + iot
            mm = p >= off
            vs = stgS[sl]
            stgS[sl] = jnp.where(mm, NN + PAD0, vs)
            vd = stgD[sl]
            stgD[sl] = jnp.where(mm, HPAD + (p & (TRASH - 1)), vd)
        pltpu.sync_copy(stgS.at[pl.ds(fb, SUP)],
                        srcPart.at[pl.ds(pbase + wp * SUP, SUP)])
        pltpu.sync_copy(stgD.at[pl.ds(fb, SUP)],
                        dstPart.at[pl.ds(pbase + wp * SUP, SUP)])
    nblk = wp + jnp.where(off > 0, 1, 0)
    blkb[pl.ds(0, LANES)] = jnp.broadcast_to(nblk, (LANES,)).astype(_i32)
    pltpu.sync_copy(blkb, blocks.at[pl.ds((c * NS + s) * LANES, LANES)])
    plsc.subcore_barrier()

    rb = c * HPAD + s * RPT
    pltpu.sync_copy(degs.at[pl.ds(rb, RPT)], dslb)
    pltpu.sync_copy(degd.at[pl.ds(rb, RPT)], ddlb)

    def scales(g, _):
        gsl = pl.ds(g * LANES, LANES)
        av = _rsqrt(dslb[gsl] + 1.0)
        bv = _rsqrt(ddlb[gsl] + 1.0)
        aslb[gsl] = av
        bslb[gsl] = bv
        abslb[gsl] = av * bv
        return 0
    lax.fori_loop(0, RPT // LANES, scales, 0)
    pltpu.sync_copy(aslb, a1.at[pl.ds(rb, RPT)])
    pltpu.sync_copy(bslb, b1.at[pl.ds(rb, RPT)])
    pltpu.sync_copy(abslb, ab1.at[pl.ds(rb, RPT)])

    def emit(k, _):
        r0 = k * DR
        pltpu.sync_copy(u0.at[pl.ds(rb + r0, DR)], u0c)
        for r in range(DR):
            sa = _splat(aslb, r0 + r)
            for q in range(D // LANES):
                qsl = pl.ds(q * LANES, LANES)
                gc[r, qsl] = u0c[r, qsl] * sa
        pltpu.sync_copy(gc, g0.at[pl.ds(rb + r0, DR)])
        return 0
    lax.fori_loop(0, NDR, emit, 0)


_LAYER_SCRATCH = (
    (pltpu.MemorySpace.VMEM_SHARED((ACC_ROWS, D), _f32),)
    + tuple(pltpu.VMEM((EC,), _i32) for _ in range(NJ))
    + tuple(pltpu.VMEM((EC,), _i32) for _ in range(NJ))
    + tuple(pltpu.VMEM((EC, D), _f32) for _ in range(NJ))
    + (pltpu.VMEM((ZC, D), _f32),
       pltpu.VMEM((LANES,), _i32),
       pltpu.SemaphoreType.DMA((NJ,)),
       pltpu.SemaphoreType.DMA((NJ,)))
)


def _zero_acc(acc, z2, s):
    def zb(k, _):
        r = k // (D // LANES)
        q = k % (D // LANES)
        z2[r, pl.ds(q * LANES, LANES)] = jnp.zeros((LANES,), _f32)
        return 0
    lax.fori_loop(0, ZC * (D // LANES), zb, 0)
    zbase = s * RPT

    def za(k, _):
        pltpu.sync_copy(z2, acc.at[pl.ds(zbase + k * ZC, ZC)])
        return 0
    lax.fori_loop(0, RPT // ZC, za, 0)


def _edge_phase(x, srcPart, dstPart, blocks, acc, spJ, dlJ, rowsJ, blkb,
                semG, semS, c, s):
    pbase = (c * NS + s) * EPT_PAD
    pltpu.sync_copy(blocks.at[pl.ds((c * NS + s) * LANES, LANES)], blkb)
    nb = jnp.max(blkb[pl.ds(0, LANES)])

    def estep(t, _):
        base = pbase + t * SUP
        gdescs = []
        for j in range(NJ):
            @pl.when(t > 0)
            def _(j=j):
                pltpu.make_async_copy(rowsJ[j], acc.at[dlJ[j]],
                                      semS.at[j]).wait()
            pltpu.sync_copy(srcPart.at[pl.ds(base + j * EC, EC)], spJ[j])
            pltpu.sync_copy(dstPart.at[pl.ds(base + j * EC, EC)], dlJ[j])
            gdescs.append(pltpu.async_copy(x.at[spJ[j]], rowsJ[j],
                                           semG.at[j]))
        for j in range(NJ):
            gdescs[j].wait()
            pltpu.async_copy(rowsJ[j], acc.at[dlJ[j]], semS.at[j], add=True)
        return 0
    lax.fori_loop(0, nb, estep, 0)

    @pl.when(nb > 0)
    def _():
        for j in range(NJ):
            pltpu.make_async_copy(rowsJ[j], acc.at[dlJ[j]],
                                  semS.at[j]).wait()


@functools.partial(
    pl.kernel,
    out_type=(
        jax.ShapeDtypeStruct((NPAD, D), _f32),
        jax.ShapeDtypeStruct((NPAD, D), _f32),
    ),
    mesh=_mesh,
    compiler_params=_params,
    scratch_types=_LAYER_SCRATCH + (
        pltpu.VMEM((RPT,), _f32),
        pltpu.VMEM((RPT,), _f32),
        pltpu.VMEM((DR, D), _f32),
        pltpu.VMEM((DR, D), _f32),
        pltpu.VMEM((DR, D), _f32),
    ),
)
def _klayer(x, b1, ab1, srcPart, dstPart, blocks, u_out, g_out, *scr):
    acc = scr[0]
    spJ = scr[1:1 + NJ]
    dlJ = scr[1 + NJ:1 + 2 * NJ]
    rowsJ = scr[1 + 2 * NJ:1 + 3 * NJ]
    z2, blkb, semG, semS, bslb, abslb, accc, uc, gc2 = scr[1 + 3 * NJ:]
    c = lax.axis_index("c")
    s = lax.axis_index("s")
    _zero_acc(acc, z2, s)
    plsc.subcore_barrier()
    _edge_phase(x, srcPart, dstPart, blocks, acc, spJ, dlJ, rowsJ, blkb,
                semG, semS, c, s)
    plsc.subcore_barrier()

    rb = c * HPAD + s * RPT
    la = s * RPT
    pltpu.sync_copy(b1.at[pl.ds(rb, RPT)], bslb)
    pltpu.sync_copy(ab1.at[pl.ds(rb, RPT)], abslb)

    def drain(k, _):
        r0 = k * DR
        pltpu.sync_copy(acc.at[pl.ds(la + r0, DR)], accc)
        for r in range(DR):
            sb = _splat(bslb, r0 + r)
            sab = _splat(abslb, r0 + r)
            for q in range(D // LANES):
                qsl = pl.ds(q * LANES, LANES)
                av = accc[r, qsl]
                uc[r, qsl] = av * sb
                gc2[r, qsl] = av * sab
        pltpu.sync_copy(uc, u_out.at[pl.ds(rb + r0, DR)])
        pltpu.sync_copy(gc2, g_out.at[pl.ds(rb + r0, DR)])
        return 0
    lax.fori_loop(0, NDR, drain, 0)


@functools.partial(
    pl.kernel,
    out_type=jax.ShapeDtypeStruct((NPAD, D), _f32),
    mesh=_mesh,
    compiler_params=_params,
    scratch_types=_LAYER_SCRATCH + (
        pltpu.VMEM((RPT,), _f32),
        pltpu.VMEM((DRF, D), _f32),
        pltpu.VMEM((DRF, D), _f32),
        pltpu.VMEM((DRF, D), _f32),
        pltpu.VMEM((DRF, D), _f32),
        pltpu.VMEM((DRF, D), _f32),
    ),
)
def _kfinal(x, b1, u0, u1, u2, srcPart, dstPart, blocks, out, *scr):
    acc = scr[0]
    spJ = scr[1:1 + NJ]
    dlJ = scr[1 + NJ:1 + 2 * NJ]
    rowsJ = scr[1 + 2 * NJ:1 + 3 * NJ]
    z2, blkb, semG, semS, bslb, accc, u0c, u1c, u2c, outc = scr[1 + 3 * NJ:]
    c = lax.axis_index("c")
    s = lax.axis_index("s")
    _zero_acc(acc, z2, s)
    plsc.subcore_barrier()
    _edge_phase(x, srcPart, dstPart, blocks, acc, spJ, dlJ, rowsJ, blkb,
                semG, semS, c, s)
    plsc.subcore_barrier()

    rb = c * HPAD + s * RPT
    la = s * RPT
    pltpu.sync_copy(b1.at[pl.ds(rb, RPT)], bslb)

    def drain(k, _):
        r0 = k * DRF
        pltpu.sync_copy(acc.at[pl.ds(la + r0, DRF)], accc)
        pltpu.sync_copy(u0.at[pl.ds(rb + r0, DRF)], u0c)
        pltpu.sync_copy(u1.at[pl.ds(rb + r0, DRF)], u1c)
        pltpu.sync_copy(u2.at[pl.ds(rb + r0, DRF)], u2c)
        for r in range(DRF):
            sb = _splat(bslb, r0 + r)
            for q in range(D // LANES):
                qsl = pl.ds(q * LANES, LANES)
                u3 = accc[r, qsl] * sb
                outc[r, qsl] = 0.25 * (u0c[r, qsl] + u1c[r, qsl]
                                       + u2c[r, qsl] + u3)
        pltpu.sync_copy(outc, out.at[pl.ds(rb + r0, DRF)])
        return 0
    lax.fori_loop(0, NDRF, drain, 0)


def kernel(user_emb, item_emb, edge_index):
    src = edge_index[0].astype(_i32)
    dst = edge_index[1].astype(_i32)
    padn = NE_PAD - NE
    pad = jnp.full((padn,), NN, _i32)
    srcp = jnp.concatenate([src, pad])
    dstp = jnp.concatenate([dst, pad])
    zpad = jnp.zeros((PAD0, D), _f32)
    u0 = jnp.concatenate([user_emb, zpad, item_emb, zpad], axis=0)

    a1, b1, ab1, g0, srcPart, dstPart, blocks = _k0(u0, srcp, dstp)
    del a1
    u1, g1 = _klayer(g0, b1, ab1, srcPart, dstPart, blocks)
    u2, g2 = _klayer(g1, b1, ab1, srcPart, dstPart, blocks)
    out = _kfinal(g2, b1, u0, u1, u2, srcPart, dstPart, blocks)

    users = out[:NU]
    items = out[HPAD:HPAD + NI]
    return (users, items)
